# baseline (speedup 1.0000x reference)
import jax
import jax.numpy as jnp
from jax import lax
from jax.experimental import pallas as pl
from jax.experimental.pallas import tpu as pltpu
from jax.experimental.pallas import tpu_sc as plsc

_CHUNK = 8
_NBUF = 12


def kernel(x, enc_weight):
    seq_len = x.shape[1]
    d = enc_weight.shape[1]
    dtype = enc_weight.dtype

    mesh = plsc.VectorSubcoreMesh(core_axis_name="c", subcore_axis_name="s")
    num_workers = mesh.num_cores * mesh.num_subcores
    rows_per_w = seq_len // num_workers
    assert rows_per_w * num_workers == seq_len
    n_chunks = rows_per_w // _CHUNK
    assert n_chunks * _CHUNK == rows_per_w and n_chunks >= _NBUF

    def body(w_hbm, o_hbm, *scratch):
        bufs = scratch[:_NBUF]
        in_sems = scratch[_NBUF:2 * _NBUF]
        out_sems = scratch[2 * _NBUF:]
        wid = lax.axis_index("s") * mesh.num_cores + lax.axis_index("c")
        base = wid * rows_per_w

        def load(i):
            return pltpu.async_copy(
                w_hbm.at[pl.ds(base + i * _CHUNK, _CHUNK)],
                bufs[i % _NBUF], in_sems[i % _NBUF])

        def store(i):
            return pltpu.async_copy(
                bufs[i % _NBUF],
                o_hbm.at[pl.ds(base + i * _CHUNK, _CHUNK)],
                out_sems[i % _NBUF])

        in_h = {}
        out_h = {}
        waited = set()
        for i in range(_NBUF - 1):
            in_h[i] = load(i)
        for i in range(n_chunks):
            j = i + _NBUF - 1
            if j < n_chunks:
                prev = j - _NBUF
                if prev >= 0:
                    out_h[prev].wait()
                    waited.add(prev)
                in_h[j] = load(j)
            in_h[i].wait()
            out_h[i] = store(i)
        for i in range(n_chunks):
            if i not in waited:
                out_h[i].wait()

    return pl.kernel(
        body,
        out_type=jax.ShapeDtypeStruct((seq_len, d), dtype),
        mesh=mesh,
        scratch_types=(
            [pltpu.VMEM((_CHUNK, d), dtype) for _ in range(_NBUF)]
            + [pltpu.SemaphoreType.DMA for _ in range(2 * _NBUF)]
        ),
    )(enc_weight)


# speedup vs baseline: 1.0214x; 1.0214x over previous
import jax
import jax.numpy as jnp
from jax import lax
from jax.experimental import pallas as pl
from jax.experimental.pallas import tpu as pltpu
from jax.experimental.pallas import tpu_sc as plsc

_CHUNK = 16
_NBUF = 7


def kernel(x, enc_weight):
    seq_len = x.shape[1]
    d = enc_weight.shape[1]
    dtype = enc_weight.dtype

    mesh = plsc.VectorSubcoreMesh(core_axis_name="c", subcore_axis_name="s")
    num_workers = mesh.num_cores * mesh.num_subcores
    rows_per_w = seq_len // num_workers
    assert rows_per_w * num_workers == seq_len
    n_chunks = rows_per_w // _CHUNK
    assert n_chunks * _CHUNK == rows_per_w and n_chunks >= _NBUF

    def body(w_hbm, o_hbm, *scratch):
        bufs = scratch[:_NBUF]
        in_sems = scratch[_NBUF:2 * _NBUF]
        out_sems = scratch[2 * _NBUF:]
        wid = lax.axis_index("s") * mesh.num_cores + lax.axis_index("c")
        base = wid * rows_per_w

        def load(i):
            return pltpu.async_copy(
                w_hbm.at[pl.ds(base + i * _CHUNK, _CHUNK)],
                bufs[i % _NBUF], in_sems[i % _NBUF])

        def store(i):
            return pltpu.async_copy(
                bufs[i % _NBUF],
                o_hbm.at[pl.ds(base + i * _CHUNK, _CHUNK)],
                out_sems[i % _NBUF])

        in_h = {}
        out_h = {}
        waited = set()
        for i in range(_NBUF - 1):
            in_h[i] = load(i)
        for i in range(n_chunks):
            j = i + _NBUF - 1
            if j < n_chunks:
                prev = j - _NBUF
                if prev >= 0:
                    out_h[prev].wait()
                    waited.add(prev)
                in_h[j] = load(j)
            in_h[i].wait()
            out_h[i] = store(i)
        for i in range(n_chunks):
            if i not in waited:
                out_h[i].wait()

    return pl.kernel(
        body,
        out_type=jax.ShapeDtypeStruct((seq_len, d), dtype),
        mesh=mesh,
        scratch_types=(
            [pltpu.VMEM((_CHUNK, d), dtype) for _ in range(_NBUF)]
            + [pltpu.SemaphoreType.DMA for _ in range(2 * _NBUF)]
        ),
    )(enc_weight)
